# baseline (speedup 1.0000x reference)
import dataclasses
import functools

import jax
import jax.numpy as jnp
from jax import lax
from jax.experimental import pallas as pl
from jax.experimental.pallas import tpu as pltpu
from jax.experimental.pallas import tpu_sc as plsc

N_PTS = 100000
N_L1 = 10000
N_L2 = 1000
E1 = 160000
E2 = 16000
D = 64
NC = 20

NCORES = 2
NSUB = 16
NW = NCORES * NSUB
LANES = 16

PTS_PAD = 100352
PTS_PER_TILE = PTS_PAD // NW
E1_PAD = 163840
NP1 = 313
L2M_PAD = 12288
NP2 = 32
E2_PAD = 16384
UNPOOL_PAD = 12288
FIN_PAD = 102400
CHUNK = 256
WIN = 128
NBUF = 4


def _sc_params():
    cp = pltpu.CompilerParams()
    fields = pltpu.CompilerParams.__dataclass_fields__
    if "needs_layout_passes" in fields:
        cp = dataclasses.replace(cp, needs_layout_passes=False)
    if "use_tc_tiling_on_sc" in fields:
        cp = dataclasses.replace(cp, use_tc_tiling_on_sc=False)
    return cp


@functools.lru_cache(maxsize=1)
def _mesh():
    return plsc.VectorSubcoreMesh(core_axis_name="c", subcore_axis_name="s")


def _wid():
    return lax.axis_index("s") * NCORES + lax.axis_index("c")


def _sc_scatter_points(p8, lab):
    @functools.partial(
        pl.kernel,
        out_type=jax.ShapeDtypeStruct((NW, N_L1 * 8), jnp.float32),
        mesh=_mesh(),
        compiler_params=_sc_params(),
        scratch_types=[
            pltpu.VMEM((PTS_PER_TILE * 8,), jnp.float32),
            pltpu.VMEM((PTS_PER_TILE,), jnp.int32),
            pltpu.VMEM((N_L1 * 8,), jnp.float32),
        ],
    )
    def k(p8_hbm, lab_hbm, out_hbm, p_v, lab_v, acc_v):
        w = _wid()
        pltpu.sync_copy(p8_hbm.at[pl.ds(w * PTS_PER_TILE * 8, PTS_PER_TILE * 8)], p_v)
        pltpu.sync_copy(lab_hbm.at[pl.ds(w * PTS_PER_TILE, PTS_PER_TILE)], lab_v)

        zeros = jnp.zeros((LANES,), jnp.float32)

        @pl.loop(0, N_L1 * 8 // LANES)
        def _(i):
            acc_v[pl.ds(i * LANES, LANES)] = zeros

        iota = lax.iota(jnp.int32, LANES)

        @pl.loop(0, PTS_PER_TILE // LANES)
        def _(g):
            labs = lab_v[pl.ds(g * LANES, LANES)]
            base8 = labs * 8
            rows8 = (iota + g * LANES) * 8
            for f in range(5):
                vals = plsc.load_gather(p_v, [rows8 + f])
                plsc.addupdate_scatter(acc_v, [base8 + f], vals)

        pltpu.sync_copy(acc_v, out_hbm.at[w])

    return k(p8, lab)


def _sc_gather(table, idx, n_out):
    d = table.shape[1]
    dt = table.dtype
    per_w = n_out // NW
    assert per_w % WIN == 0
    nwin = per_w // WIN
    ngrp, tail = nwin // NBUF, nwin % NBUF

    @functools.partial(
        pl.kernel,
        out_type=jax.ShapeDtypeStruct((n_out, d), dt),
        mesh=_mesh(),
        compiler_params=_sc_params(),
        scratch_types=[
            pltpu.VMEM((per_w,), jnp.int32),
            pltpu.VMEM((NBUF * WIN, d), dt),
            pltpu.SemaphoreType.DMA,
        ],
    )
    def k(t_hbm, i_hbm, o_hbm, idx_v, rows_v, sem):
        w = _wid()
        base = w * per_w
        pltpu.sync_copy(i_hbm.at[pl.ds(base, per_w)], idx_v)

        def group(g0, k_):
            cps = []
            for b in range(k_):
                cps.append(pltpu.async_copy(
                    t_hbm.at[idx_v.at[pl.ds((g0 + b) * WIN, WIN)]],
                    rows_v.at[pl.ds(b * WIN, WIN)], sem))
            for cp in cps:
                cp.wait()
            pltpu.sync_copy(rows_v.at[pl.ds(0, k_ * WIN)],
                            o_hbm.at[pl.ds(base + g0 * WIN, k_ * WIN)])

        if ngrp:
            @pl.loop(0, ngrp)
            def _(gg):
                group(gg * NBUF, NBUF)
        if tail:
            group(ngrp * NBUF, tail)

    return k(table, idx)


EPT = E1_PAD // NW


def _sc_rel(c4flat, src, dst):
    @functools.partial(
        pl.kernel,
        out_type=jax.ShapeDtypeStruct((E1_PAD * 4,), jnp.float32),
        mesh=_mesh(),
        compiler_params=_sc_params(),
        scratch_types=[
            pltpu.VMEM((N_L1 * 4,), jnp.float32),
            pltpu.VMEM((EPT,), jnp.int32),
            pltpu.VMEM((EPT,), jnp.int32),
            pltpu.VMEM((EPT * 4,), jnp.float32),
        ],
    )
    def k(c_hbm, s_hbm, d_hbm, o_hbm, c_v, s_v, d_v, o_v):
        w = _wid()
        pltpu.sync_copy(c_hbm, c_v)
        pltpu.sync_copy(s_hbm.at[pl.ds(w * EPT, EPT)], s_v)
        pltpu.sync_copy(d_hbm.at[pl.ds(w * EPT, EPT)], d_v)

        zeros = jnp.zeros((LANES,), jnp.float32)

        @pl.loop(0, EPT * 4 // LANES)
        def _(i):
            o_v[pl.ds(i * LANES, LANES)] = zeros

        iota4 = lax.iota(jnp.int32, LANES) * 4

        @pl.loop(0, EPT // LANES)
        def _(g):
            s4 = s_v[pl.ds(g * LANES, LANES)] * 4
            d4 = d_v[pl.ds(g * LANES, LANES)] * 4
            pos = iota4 + g * (LANES * 4)
            for c in range(3):
                a = plsc.load_gather(c_v, [s4 + c])
                b = plsc.load_gather(c_v, [d4 + c])
                plsc.store_scatter(o_v, [pos + c], a - b)

        pltpu.sync_copy(o_v, o_hbm.at[pl.ds(w * EPT * 4, EPT * 4)])

    return k(c4flat, src, dst).reshape(E1_PAD, 4)


def _sc_segmax(vals, dst, bounds, e_pad, npt):
    n_out = NW * npt

    @functools.partial(
        pl.kernel,
        out_type=jax.ShapeDtypeStruct((n_out * D,), jnp.float32),
        mesh=_mesh(),
        compiler_params=_sc_params(),
        scratch_types=[
            pltpu.VMEM((npt * D,), jnp.float32),
            pltpu.VMEM((CHUNK, D), jnp.float32),
            pltpu.VMEM((CHUNK,), jnp.int32),
            pltpu.VMEM((48,), jnp.int32),
        ],
    )
    def k(v_hbm, d_hbm, b_hbm, o_hbm, acc_v, buf_v, dbuf_v, b_v):
        w = _wid()
        pltpu.sync_copy(b_hbm, b_v)
        wsplat = jnp.full((LANES,), w, jnp.int32)
        lo = plsc.load_gather(b_v, [wsplat])[0]
        hi = plsc.load_gather(b_v, [wsplat + 1])[0]
        base = w * npt

        ninf = jnp.full((LANES,), -jnp.inf, jnp.float32)

        @pl.loop(0, npt * D // LANES)
        def _(i):
            acc_v[pl.ds(i * LANES, LANES)] = ninf

        k0 = lo // CHUNK
        nchunks = (hi + CHUNK - 1) // CHUNK - k0

        @pl.loop(0, nchunks)
        def _(kk):
            start = (k0 + kk) * CHUNK
            pltpu.sync_copy(v_hbm.at[pl.ds(start, CHUNK)], buf_v)
            pltpu.sync_copy(d_hbm.at[pl.ds(start, CHUNK)], dbuf_v)

            @pl.loop(0, CHUNK // LANES)
            def _(g):
                dvec = dbuf_v[pl.ds(g * LANES, LANES)]
                for l in range(LANES):
                    i = g * LANES + l
                    gi = start + i
                    ok = jnp.logical_and(gi >= lo, gi < hi)
                    dl = jnp.where(ok, dvec[l] - base, 0)
                    for j in range(D // LANES):
                        v = buf_v[i, pl.ds(j * LANES, LANES)]
                        v = jnp.where(ok, v, -jnp.inf)
                        off = dl * D + j * LANES
                        acc_v[pl.ds(off, LANES)] = jnp.maximum(
                            acc_v[pl.ds(off, LANES)], v)

        pltpu.sync_copy(acc_v, o_hbm.at[pl.ds(base * D, npt * D)])

    return k(vals, dst, bounds)


def _full(a):
    return pl.BlockSpec(a.shape, lambda *_: tuple(0 for _ in a.shape))


def _tc_combine_mlp1(part, l1c4, W1, b1, W2, b2, W1x):
    B = 1000

    def f(p_ref, c_ref, w1_ref, b1_ref, w2_ref, b2_ref, wx_ref, t1_ref, hx_ref):
        S = jnp.sum(p_ref[...], axis=0)
        agg = jnp.concatenate(
            [S[:, 0:1], S[:, 1:4] - S[:, 4:5] * c_ref[...][:, 0:3]], axis=1)
        h = jnp.maximum(jnp.dot(agg, w1_ref[...],
                                preferred_element_type=jnp.float32) + b1_ref[...], 0.0)
        t1 = jnp.dot(h, w2_ref[...], preferred_element_type=jnp.float32) + b2_ref[...]
        t1_ref[...] = t1
        hx_ref[...] = jnp.dot(t1, wx_ref[...], preferred_element_type=jnp.float32)

    return pl.pallas_call(
        f,
        grid=(N_L1 // B,),
        in_specs=[
            pl.BlockSpec((NW, B, 8), lambda i: (0, i, 0)),
            pl.BlockSpec((B, 4), lambda i: (i, 0)),
            _full(W1), _full(b1), _full(W2), _full(b2), _full(W1x),
        ],
        out_specs=[pl.BlockSpec((B, D), lambda i: (i, 0)),
                   pl.BlockSpec((B, D), lambda i: (i, 0))],
        out_shape=[jax.ShapeDtypeStruct((N_L1, D), jnp.float32),
                   jax.ShapeDtypeStruct((N_L1, D), jnp.float32)],
    )(part, l1c4, W1, b1, W2, b2, W1x)


def _pack32(h):
    lo = lax.bitcast_convert_type(h[:, :32].astype(jnp.bfloat16),
                                  jnp.uint16).astype(jnp.uint32)
    hi = lax.bitcast_convert_type(h[:, 32:].astype(jnp.bfloat16),
                                  jnp.uint16).astype(jnp.uint32)
    return lax.bitcast_convert_type(lo | (hi << 16), jnp.int32)


def _unpack64(xi):
    lo = lax.bitcast_convert_type(jnp.left_shift(xi, 16), jnp.float32)
    hi = lax.bitcast_convert_type(
        jnp.bitwise_and(xi, jnp.int32(-65536)), jnp.float32)
    return jnp.concatenate([lo, hi], axis=1)


def _tc_edge(hsrc_p, rel4, W1r4, b1, W2):
    B = 1024

    def f(h_ref, r_ref, wr_ref, b1_ref, w2_ref, o_ref):
        pre = _unpack64(h_ref[...]) + jnp.dot(
            r_ref[...], wr_ref[...],
            preferred_element_type=jnp.float32) + b1_ref[...]
        o_ref[...] = jnp.dot(jnp.maximum(pre, 0.0), w2_ref[...],
                             preferred_element_type=jnp.float32)

    return pl.pallas_call(
        f,
        grid=(E1_PAD // B,),
        in_specs=[pl.BlockSpec((B, D // 2), lambda i: (i, 0)),
                  pl.BlockSpec((B, 4), lambda i: (i, 0)),
                  _full(W1r4), _full(b1), _full(W2)],
        out_specs=pl.BlockSpec((B, D), lambda i: (i, 0)),
        out_shape=jax.ShapeDtypeStruct((E1_PAD, D), jnp.float32),
    )(hsrc_p, rel4, W1r4, b1, W2)


def _tc_node(raw, x, b2e, Wo1, bo1, Wo2, bo2, skip=None, Wxn=None):
    B = 1000
    n_in = [raw, x, b2e, Wo1, bo1, Wo2, bo2]
    specs = [pl.BlockSpec((B, D), lambda i: (i, 0)),
             pl.BlockSpec((B, D), lambda i: (i, 0)),
             _full(b2e), _full(Wo1), _full(bo1), _full(Wo2), _full(bo2)]
    if skip is not None:
        n_in.append(skip)
        specs.append(pl.BlockSpec((B, D), lambda i: (i, 0)))
    if Wxn is not None:
        n_in.append(Wxn)
        specs.append(_full(Wxn))

    has_skip = skip is not None
    has_wxn = Wxn is not None

    def f(*refs):
        raw_r, x_r, b2_r, w1_r, b1_r, w2_r, bo2_r = refs[:7]
        pos = 7
        rawv = raw_r[...]
        agg = jnp.where(jnp.isfinite(rawv), rawv + b2_r[...], 0.0)
        h = jnp.maximum(jnp.dot(agg, w1_r[...],
                                preferred_element_type=jnp.float32) + b1_r[...], 0.0)
        y = x_r[...] + jnp.dot(h, w2_r[...],
                               preferred_element_type=jnp.float32) + bo2_r[...]
        if has_skip:
            y = y + refs[pos][...]
            pos += 1
        if has_wxn:
            wx_r = refs[pos]
            y_ref, hx_ref = refs[-2:]
            y_ref[...] = y
            hx_ref[...] = jnp.dot(y, wx_r[...], preferred_element_type=jnp.float32)
        else:
            refs[-1][...] = y

    out_specs = pl.BlockSpec((B, D), lambda i: (i, 0))
    out_shape = jax.ShapeDtypeStruct((N_L1, D), jnp.float32)
    if has_wxn:
        out_specs = [out_specs, pl.BlockSpec((B, D), lambda i: (i, 0))]
        out_shape = [out_shape, jax.ShapeDtypeStruct((N_L1, D), jnp.float32)]

    return pl.pallas_call(
        f,
        grid=(N_L1 // B,),
        in_specs=specs,
        out_specs=out_specs,
        out_shape=out_shape,
    )(*n_in)


def _tc_matmul(x, W, n_rows):
    B = 1000

    def f(x_ref, w_ref, o_ref):
        o_ref[...] = jnp.dot(x_ref[...], w_ref[...],
                             preferred_element_type=jnp.float32)

    return pl.pallas_call(
        f,
        grid=(n_rows // B,),
        in_specs=[pl.BlockSpec((B, D), lambda i: (i, 0)), _full(W)],
        out_specs=pl.BlockSpec((B, D), lambda i: (i, 0)),
        out_shape=jax.ShapeDtypeStruct((n_rows, D), jnp.float32),
    )(x, W)


def _tc_finite0(raw, add=None):
    has_add = add is not None

    def f(*refs):
        r = refs[0][...]
        t = jnp.where(jnp.isfinite(r), r, 0.0)
        if has_add:
            t = t + refs[1][...]
        refs[-1][...] = t

    n_in = [raw] + ([add] if has_add else [])
    specs = [pl.BlockSpec((N_L2, D), lambda: (0, 0))]
    if has_add:
        specs.append(pl.BlockSpec((N_L2, D), lambda: (0, 0)))
    return pl.pallas_call(
        f,
        grid=(),
        in_specs=specs,
        out_specs=pl.BlockSpec((N_L2, D), lambda: (0, 0)),
        out_shape=jax.ShapeDtypeStruct((N_L2, D), jnp.float32),
    )(*n_in)


def _tc_vtable(t6, l1c4, W1a, W1p4, b1f):
    B = 1000

    def f(t_ref, c_ref, wa_ref, wp_ref, b_ref, o_ref):
        o_ref[...] = (jnp.dot(t_ref[...], wa_ref[...],
                              preferred_element_type=jnp.float32)
                      - jnp.dot(c_ref[...], wp_ref[...],
                                preferred_element_type=jnp.float32)
                      + b_ref[...])

    return pl.pallas_call(
        f,
        grid=(N_L1 // B,),
        in_specs=[pl.BlockSpec((B, D), lambda i: (i, 0)),
                  pl.BlockSpec((B, 4), lambda i: (i, 0)),
                  _full(W1a), _full(W1p4), _full(b1f)],
        out_specs=pl.BlockSpec((B, D), lambda i: (i, 0)),
        out_shape=jax.ShapeDtypeStruct((N_L1, D), jnp.float32),
    )(t6, l1c4, W1a, W1p4, b1f)


def _tc_final(vg_p, P4, Wp, W2f, b2f, Wc, bc):
    B = 2000

    def f(v_ref, p_ref, wp_ref, w2_ref, b2_ref, wc_ref, bc_ref, o_ref):
        pre = _unpack64(v_ref[...]) + jnp.dot(p_ref[...], wp_ref[...],
                                              preferred_element_type=jnp.float32)
        h = jnp.maximum(pre, 0.0)
        t7 = jnp.dot(h, w2_ref[...], preferred_element_type=jnp.float32) + b2_ref[...]
        o_ref[...] = jnp.dot(t7, wc_ref[...],
                             preferred_element_type=jnp.float32) + bc_ref[...]

    return pl.pallas_call(
        f,
        grid=(N_PTS // B,),
        in_specs=[pl.BlockSpec((B, D // 2), lambda i: (i, 0)),
                  pl.BlockSpec((B, 4), lambda i: (i, 0)),
                  _full(Wp), _full(W2f), _full(b2f), _full(Wc), _full(bc)],
        out_specs=pl.BlockSpec((B, NC), lambda i: (i, 0)),
        out_shape=jax.ShapeDtypeStruct((N_PTS, NC), jnp.float32),
    )(vg_p, P4, Wp, W2f, b2f, Wc, bc)


def _pad_rows(a, n):
    return jnp.pad(a, ((0, n - a.shape[0]), (0, 0)))


def _pad1(a, n, val=0):
    return jnp.pad(a, (0, n - a.shape[0]), constant_values=val)


def _bounds(sorted_ids, npt):
    cuts = (jnp.arange(33, dtype=jnp.int32) * npt).astype(jnp.int32)
    b = jnp.searchsorted(sorted_ids, cuts, side="left").astype(jnp.int32)
    return _pad1(b, 48)


def kernel(remission, points, l1_cluster_centers, l2_cluster_centers,
           l1_edges, l2_edges, l1_labels, l2_labels, params):
    f32 = jnp.float32
    l1c = l1_cluster_centers.astype(f32)
    l1c4 = jnp.pad(l1c, ((0, 0), (0, 1)))

    src = l1_edges[0].astype(jnp.int32)
    dst = l1_edges[1].astype(jnp.int32)
    perm1 = jnp.argsort(dst)
    src_s = _pad1(src[perm1], E1_PAD)
    dst_s = _pad1(dst[perm1], E1_PAD, val=NW * NP1 + 7)
    bounds1 = _bounds(dst_s[:E1], NP1)

    lab2 = l2_labels.astype(jnp.int32)
    perm_l2 = jnp.argsort(lab2)
    l2srt = lab2[perm_l2]
    perm_l2p = _pad1(perm_l2, L2M_PAD)
    l2srt_p = _pad1(l2srt, L2M_PAD, val=NW * NP2 + 7)
    bounds2 = _bounds(l2srt, NP2)

    src4 = l2_edges[0].astype(jnp.int32)
    dst4 = l2_edges[1].astype(jnp.int32)
    perm4 = jnp.argsort(dst4)
    src4_s = _pad1(src4[perm4], E2_PAD)
    dst4_s = _pad1(dst4[perm4], E2_PAD, val=NW * NP2 + 7)
    bounds4 = _bounds(dst4_s[:E2], NP2)

    lab1 = l1_labels.astype(jnp.int32)
    lab1_p = _pad1(lab1, PTS_PAD)
    lab1_fin = _pad1(lab1, FIN_PAD)
    l2lab_p = _pad1(lab2, UNPOOL_PAD)

    P4 = jnp.concatenate([remission.astype(f32), points.astype(f32)], axis=1)
    P8 = jnp.concatenate(
        [P4, jnp.ones((N_PTS, 1), f32), jnp.zeros((N_PTS, 3), f32)], axis=1)
    P8 = _pad_rows(P8, PTS_PAD).reshape(-1)

    p = params
    ffn = p["ffn"]

    def b2d(b):
        return b.reshape(1, -1).astype(f32)

    part = _sc_scatter_points(P8, lab1_p).reshape(NW, N_L1, 8)
    names = ["g2", "g2_1", "g2_2", "g2_3", "g6", "g6_1", "g6_2", "g6_3"]
    W1x = {n: p[n]["edge"]["W1"][:D].astype(f32) for n in names}
    W1r4 = {n: jnp.pad(p[n]["edge"]["W1"][D:].astype(f32), ((0, 1), (0, 0)))
            for n in names}
    t1, hx = _tc_combine_mlp1(part, l1c4, ffn["W1"].astype(f32), b2d(ffn["b1"]),
                              ffn["W2"].astype(f32), b2d(ffn["b2"]), W1x["g2"])

    rel4 = _sc_rel(l1c4.reshape(-1), src_s, jnp.clip(dst_s, 0, N_L1 - 1))

    def gnn(x, hx, name, skip=None, next_name=None):
        pe, po = p[name]["edge"], p[name]["out"]
        hsrc_p = _sc_gather(_pack32(hx), src_s, E1_PAD)
        ef = _tc_edge(hsrc_p, rel4, W1r4[name], b2d(pe["b1"]), pe["W2"].astype(f32))
        raw = _sc_segmax(ef, dst_s, bounds1, E1_PAD, NP1).reshape(-1, D)
        wxn = W1x[next_name] if next_name else None
        return _tc_node(raw[:N_L1], x, b2d(pe["b2"]), po["W1"].astype(f32),
                        b2d(po["b1"]), po["W2"].astype(f32), b2d(po["b2"]),
                        skip=skip, Wxn=wxn)

    t2, hx = gnn(t1, hx, "g2", next_name="g2_1")
    t2_1, hx = gnn(t2, hx, "g2_1", next_name="g2_2")
    t2_2, hx = gnn(t2_1, hx, "g2_2", next_name="g2_3")
    t2_3 = gnn(t2_2, hx, "g2_3")

    g23s = _sc_gather(t2_3, perm_l2p, L2M_PAD)
    raw3 = _sc_segmax(g23s, l2srt_p, bounds2, L2M_PAD, NP2).reshape(-1, D)
    t3 = _tc_finite0(raw3[:N_L2])
    g3 = _sc_gather(t3, src4_s, E2_PAD)
    raw4 = _sc_segmax(g3, dst4_s, bounds4, E2_PAD, NP2).reshape(-1, D)
    t4 = _tc_finite0(raw4[:N_L2], add=t3)
    t5 = _sc_gather(t4, l2lab_p, UNPOOL_PAD)[:N_L1]
    hx = _tc_matmul(t5, W1x["g6"], N_L1)

    t6, hx = gnn(t5, hx, "g6", skip=t2_3, next_name="g6_1")
    t6, hx = gnn(t6, hx, "g6_1", skip=t2_2, next_name="g6_2")
    t6, hx = gnn(t6, hx, "g6_2", skip=t2_1, next_name="g6_3")
    t6 = gnn(t6, hx, "g6_3", skip=t2)

    fb = p["fbn"]
    W1a = fb["W1"][:D].astype(f32)
    W1p = fb["W1"][D:D + 3].astype(f32)
    w1r = fb["W1"][D + 3].astype(f32)
    W1p4 = jnp.pad(W1p, ((0, 1), (0, 0)))
    v = _tc_vtable(t6, l1c4, W1a, W1p4, b2d(fb["b1"]))
    vg_p = _sc_gather(_pack32(v), lab1_fin, FIN_PAD)
    Wp = jnp.concatenate([w1r[None, :], W1p], axis=0)
    logits = _tc_final(vg_p[:N_PTS], P4, Wp, fb["W2"].astype(f32), b2d(fb["b2"]),
                       p["cls"]["W"].astype(f32), b2d(p["cls"]["b"]))
    return logits


# speedup vs baseline: 1.0746x; 1.0746x over previous
import dataclasses
import functools

import jax
import jax.numpy as jnp
from jax import lax
from jax.experimental import pallas as pl
from jax.experimental.pallas import tpu as pltpu
from jax.experimental.pallas import tpu_sc as plsc

N_PTS = 100000
N_L1 = 10000
N_L2 = 1000
E1 = 160000
E2 = 16000
D = 64
NC = 20

NCORES = 2
NSUB = 16
NW = NCORES * NSUB
LANES = 16

PTS_PAD = 100352
PTS_PER_TILE = PTS_PAD // NW
E1_PAD = 160768
NP1 = 313
L2M_PAD = 10240
NP2 = 32
E2_PAD = 16128
UNPOOL_PAD = 10112
FIN_PAD = 100096
CHUNK = 256


def _sc_params():
    cp = pltpu.CompilerParams()
    fields = pltpu.CompilerParams.__dataclass_fields__
    if "needs_layout_passes" in fields:
        cp = dataclasses.replace(cp, needs_layout_passes=False)
    if "use_tc_tiling_on_sc" in fields:
        cp = dataclasses.replace(cp, use_tc_tiling_on_sc=False)
    return cp


@functools.lru_cache(maxsize=1)
def _mesh():
    return plsc.VectorSubcoreMesh(core_axis_name="c", subcore_axis_name="s")


def _wid():
    return lax.axis_index("s") * NCORES + lax.axis_index("c")


def _sc_scatter_points(p8, lab):
    @functools.partial(
        pl.kernel,
        out_type=jax.ShapeDtypeStruct((NW, N_L1 * 8), jnp.float32),
        mesh=_mesh(),
        compiler_params=_sc_params(),
        scratch_types=[
            pltpu.VMEM((PTS_PER_TILE * 8,), jnp.float32),
            pltpu.VMEM((PTS_PER_TILE,), jnp.int32),
            pltpu.VMEM((N_L1 * 8,), jnp.float32),
        ],
    )
    def k(p8_hbm, lab_hbm, out_hbm, p_v, lab_v, acc_v):
        w = _wid()
        pltpu.sync_copy(p8_hbm.at[pl.ds(w * PTS_PER_TILE * 8, PTS_PER_TILE * 8)], p_v)
        pltpu.sync_copy(lab_hbm.at[pl.ds(w * PTS_PER_TILE, PTS_PER_TILE)], lab_v)

        zeros = jnp.zeros((LANES,), jnp.float32)

        @pl.loop(0, N_L1 * 8 // LANES)
        def _(i):
            acc_v[pl.ds(i * LANES, LANES)] = zeros

        iota = lax.iota(jnp.int32, LANES)

        @pl.loop(0, PTS_PER_TILE // LANES)
        def _(g):
            labs = lab_v[pl.ds(g * LANES, LANES)]
            base8 = labs * 8
            rows8 = (iota + g * LANES) * 8
            for f in range(5):
                vals = plsc.load_gather(p_v, [rows8 + f])
                plsc.addupdate_scatter(acc_v, [base8 + f], vals)

        pltpu.sync_copy(acc_v, out_hbm.at[w])

    return k(p8, lab)


def _sc_gather(table, idx, n_out):
    idx2 = idx.reshape(1, n_out)
    d = table.shape[1]
    dt = table.dtype

    @functools.partial(
        pl.kernel,
        out_type=jax.ShapeDtypeStruct((n_out, d), dt),
        mesh=_mesh(),
        compiler_params=_sc_params(),
    )
    def k(t_hbm, i_hbm, o_hbm):
        def body(i_vmem, o_vmem):
            pltpu.sync_copy(t_hbm.at[i_vmem.at[0]], o_vmem)

        pltpu.emit_pipeline(
            body,
            grid=(n_out // 128,),
            in_specs=[pl.BlockSpec((1, 128), index_map=lambda i: (0, i))],
            out_specs=[pl.BlockSpec((128, d), index_map=lambda i: (i, 0))],
            core_axis_name=("c", "s"),
            dimension_semantics=(pltpu.PARALLEL,),
        )(i_hbm, o_hbm)

    return k(table, idx2)


EPT = E1_PAD // NW


def _sc_rel(c4flat, src, dst):
    @functools.partial(
        pl.kernel,
        out_type=jax.ShapeDtypeStruct((E1_PAD * 4,), jnp.float32),
        mesh=_mesh(),
        compiler_params=_sc_params(),
        scratch_types=[
            pltpu.VMEM((N_L1 * 4,), jnp.float32),
            pltpu.VMEM((EPT,), jnp.int32),
            pltpu.VMEM((EPT,), jnp.int32),
            pltpu.VMEM((EPT * 4,), jnp.float32),
        ],
    )
    def k(c_hbm, s_hbm, d_hbm, o_hbm, c_v, s_v, d_v, o_v):
        w = _wid()
        pltpu.sync_copy(c_hbm, c_v)
        pltpu.sync_copy(s_hbm.at[pl.ds(w * EPT, EPT)], s_v)
        pltpu.sync_copy(d_hbm.at[pl.ds(w * EPT, EPT)], d_v)

        zeros = jnp.zeros((LANES,), jnp.float32)

        @pl.loop(0, EPT * 4 // LANES)
        def _(i):
            o_v[pl.ds(i * LANES, LANES)] = zeros

        iota4 = lax.iota(jnp.int32, LANES) * 4

        @pl.loop(0, EPT // LANES)
        def _(g):
            s4 = s_v[pl.ds(g * LANES, LANES)] * 4
            d4 = d_v[pl.ds(g * LANES, LANES)] * 4
            pos = iota4 + g * (LANES * 4)
            for c in range(3):
                a = plsc.load_gather(c_v, [s4 + c])
                b = plsc.load_gather(c_v, [d4 + c])
                plsc.store_scatter(o_v, [pos + c], a - b)

        pltpu.sync_copy(o_v, o_hbm.at[pl.ds(w * EPT * 4, EPT * 4)])

    return k(c4flat, src, dst).reshape(E1_PAD, 4)


def _sc_segmax(vals, dst, bounds, e_pad, npt):
    n_out = NW * npt

    @functools.partial(
        pl.kernel,
        out_type=jax.ShapeDtypeStruct((n_out * D,), jnp.float32),
        mesh=_mesh(),
        compiler_params=_sc_params(),
        scratch_types=[
            pltpu.VMEM((npt * D,), jnp.float32),
            pltpu.VMEM((CHUNK, D), jnp.float32),
            pltpu.VMEM((CHUNK,), jnp.int32),
            pltpu.VMEM((48,), jnp.int32),
        ],
    )
    def k(v_hbm, d_hbm, b_hbm, o_hbm, acc_v, buf_v, dbuf_v, b_v):
        w = _wid()
        pltpu.sync_copy(b_hbm, b_v)
        wsplat = jnp.full((LANES,), w, jnp.int32)
        lo = plsc.load_gather(b_v, [wsplat])[0]
        hi = plsc.load_gather(b_v, [wsplat + 1])[0]
        base = w * npt

        ninf = jnp.full((LANES,), -jnp.inf, jnp.float32)

        @pl.loop(0, npt * D // LANES)
        def _(i):
            acc_v[pl.ds(i * LANES, LANES)] = ninf

        k0 = lo // CHUNK
        nchunks = (hi + CHUNK - 1) // CHUNK - k0

        @pl.loop(0, nchunks)
        def _(kk):
            start = (k0 + kk) * CHUNK
            pltpu.sync_copy(v_hbm.at[pl.ds(start, CHUNK)], buf_v)
            pltpu.sync_copy(d_hbm.at[pl.ds(start, CHUNK)], dbuf_v)

            @pl.loop(0, CHUNK // LANES)
            def _(g):
                dvec = dbuf_v[pl.ds(g * LANES, LANES)]
                for l in range(LANES):
                    i = g * LANES + l
                    gi = start + i
                    ok = jnp.logical_and(gi >= lo, gi < hi)
                    dl = jnp.where(ok, dvec[l] - base, 0)
                    for j in range(D // LANES):
                        v = buf_v[i, pl.ds(j * LANES, LANES)]
                        v = jnp.where(ok, v, -jnp.inf)
                        off = dl * D + j * LANES
                        acc_v[pl.ds(off, LANES)] = jnp.maximum(
                            acc_v[pl.ds(off, LANES)], v)

        pltpu.sync_copy(acc_v, o_hbm.at[pl.ds(base * D, npt * D)])

    return k(vals, dst, bounds)


def _full(a):
    return pl.BlockSpec(a.shape, lambda *_: tuple(0 for _ in a.shape))


def _tc_combine_mlp1(part, l1c4, W1, b1, W2, b2, W1x):
    B = 1000

    def f(p_ref, c_ref, w1_ref, b1_ref, w2_ref, b2_ref, wx_ref, t1_ref, hx_ref):
        S = jnp.sum(p_ref[...], axis=0)
        agg = jnp.concatenate(
            [S[:, 0:1], S[:, 1:4] - S[:, 4:5] * c_ref[...][:, 0:3]], axis=1)
        h = jnp.maximum(jnp.dot(agg, w1_ref[...],
                                preferred_element_type=jnp.float32) + b1_ref[...], 0.0)
        t1 = jnp.dot(h, w2_ref[...], preferred_element_type=jnp.float32) + b2_ref[...]
        t1_ref[...] = t1
        hx_ref[...] = jnp.dot(t1, wx_ref[...], preferred_element_type=jnp.float32)

    return pl.pallas_call(
        f,
        grid=(N_L1 // B,),
        in_specs=[
            pl.BlockSpec((NW, B, 8), lambda i: (0, i, 0)),
            pl.BlockSpec((B, 4), lambda i: (i, 0)),
            _full(W1), _full(b1), _full(W2), _full(b2), _full(W1x),
        ],
        out_specs=[pl.BlockSpec((B, D), lambda i: (i, 0)),
                   pl.BlockSpec((B, D), lambda i: (i, 0))],
        out_shape=[jax.ShapeDtypeStruct((N_L1, D), jnp.float32),
                   jax.ShapeDtypeStruct((N_L1, D), jnp.float32)],
    )(part, l1c4, W1, b1, W2, b2, W1x)


def _pack32(h):
    lo = lax.bitcast_convert_type(h[:, :32].astype(jnp.bfloat16),
                                  jnp.uint16).astype(jnp.uint32)
    hi = lax.bitcast_convert_type(h[:, 32:].astype(jnp.bfloat16),
                                  jnp.uint16).astype(jnp.uint32)
    return lax.bitcast_convert_type(lo | (hi << 16), jnp.int32)


def _unpack64(xi):
    lo = lax.bitcast_convert_type(jnp.left_shift(xi, 16), jnp.float32)
    hi = lax.bitcast_convert_type(
        jnp.bitwise_and(xi, jnp.int32(-65536)), jnp.float32)
    return jnp.concatenate([lo, hi], axis=1)


def _tc_edge(hsrc_p, rel4, W1r4, b1, W2):
    B = 1024

    def f(h_ref, r_ref, wr_ref, b1_ref, w2_ref, o_ref):
        pre = _unpack64(h_ref[...]) + jnp.dot(
            r_ref[...], wr_ref[...],
            preferred_element_type=jnp.float32) + b1_ref[...]
        o_ref[...] = jnp.dot(jnp.maximum(pre, 0.0), w2_ref[...],
                             preferred_element_type=jnp.float32)

    return pl.pallas_call(
        f,
        grid=(E1_PAD // B,),
        in_specs=[pl.BlockSpec((B, D // 2), lambda i: (i, 0)),
                  pl.BlockSpec((B, 4), lambda i: (i, 0)),
                  _full(W1r4), _full(b1), _full(W2)],
        out_specs=pl.BlockSpec((B, D), lambda i: (i, 0)),
        out_shape=jax.ShapeDtypeStruct((E1_PAD, D), jnp.float32),
    )(hsrc_p, rel4, W1r4, b1, W2)


def _tc_node(raw, x, b2e, Wo1, bo1, Wo2, bo2, skip=None, Wxn=None):
    B = 1000
    n_in = [raw, x, b2e, Wo1, bo1, Wo2, bo2]
    specs = [pl.BlockSpec((B, D), lambda i: (i, 0)),
             pl.BlockSpec((B, D), lambda i: (i, 0)),
             _full(b2e), _full(Wo1), _full(bo1), _full(Wo2), _full(bo2)]
    if skip is not None:
        n_in.append(skip)
        specs.append(pl.BlockSpec((B, D), lambda i: (i, 0)))
    if Wxn is not None:
        n_in.append(Wxn)
        specs.append(_full(Wxn))

    has_skip = skip is not None
    has_wxn = Wxn is not None

    def f(*refs):
        raw_r, x_r, b2_r, w1_r, b1_r, w2_r, bo2_r = refs[:7]
        pos = 7
        rawv = raw_r[...]
        agg = jnp.where(jnp.isfinite(rawv), rawv + b2_r[...], 0.0)
        h = jnp.maximum(jnp.dot(agg, w1_r[...],
                                preferred_element_type=jnp.float32) + b1_r[...], 0.0)
        y = x_r[...] + jnp.dot(h, w2_r[...],
                               preferred_element_type=jnp.float32) + bo2_r[...]
        if has_skip:
            y = y + refs[pos][...]
            pos += 1
        if has_wxn:
            wx_r = refs[pos]
            y_ref, hx_ref = refs[-2:]
            y_ref[...] = y
            hx_ref[...] = jnp.dot(y, wx_r[...], preferred_element_type=jnp.float32)
        else:
            refs[-1][...] = y

    out_specs = pl.BlockSpec((B, D), lambda i: (i, 0))
    out_shape = jax.ShapeDtypeStruct((N_L1, D), jnp.float32)
    if has_wxn:
        out_specs = [out_specs, pl.BlockSpec((B, D), lambda i: (i, 0))]
        out_shape = [out_shape, jax.ShapeDtypeStruct((N_L1, D), jnp.float32)]

    return pl.pallas_call(
        f,
        grid=(N_L1 // B,),
        in_specs=specs,
        out_specs=out_specs,
        out_shape=out_shape,
    )(*n_in)


def _tc_matmul(x, W, n_rows):
    B = 1000

    def f(x_ref, w_ref, o_ref):
        o_ref[...] = jnp.dot(x_ref[...], w_ref[...],
                             preferred_element_type=jnp.float32)

    return pl.pallas_call(
        f,
        grid=(n_rows // B,),
        in_specs=[pl.BlockSpec((B, D), lambda i: (i, 0)), _full(W)],
        out_specs=pl.BlockSpec((B, D), lambda i: (i, 0)),
        out_shape=jax.ShapeDtypeStruct((n_rows, D), jnp.float32),
    )(x, W)


def _tc_finite0(raw, add=None):
    has_add = add is not None

    def f(*refs):
        r = refs[0][...]
        t = jnp.where(jnp.isfinite(r), r, 0.0)
        if has_add:
            t = t + refs[1][...]
        refs[-1][...] = t

    n_in = [raw] + ([add] if has_add else [])
    specs = [pl.BlockSpec((N_L2, D), lambda: (0, 0))]
    if has_add:
        specs.append(pl.BlockSpec((N_L2, D), lambda: (0, 0)))
    return pl.pallas_call(
        f,
        grid=(),
        in_specs=specs,
        out_specs=pl.BlockSpec((N_L2, D), lambda: (0, 0)),
        out_shape=jax.ShapeDtypeStruct((N_L2, D), jnp.float32),
    )(*n_in)


def _tc_vtable(t6, l1c4, W1a, W1p4, b1f):
    B = 1000

    def f(t_ref, c_ref, wa_ref, wp_ref, b_ref, o_ref):
        o_ref[...] = (jnp.dot(t_ref[...], wa_ref[...],
                              preferred_element_type=jnp.float32)
                      - jnp.dot(c_ref[...], wp_ref[...],
                                preferred_element_type=jnp.float32)
                      + b_ref[...])

    return pl.pallas_call(
        f,
        grid=(N_L1 // B,),
        in_specs=[pl.BlockSpec((B, D), lambda i: (i, 0)),
                  pl.BlockSpec((B, 4), lambda i: (i, 0)),
                  _full(W1a), _full(W1p4), _full(b1f)],
        out_specs=pl.BlockSpec((B, D), lambda i: (i, 0)),
        out_shape=jax.ShapeDtypeStruct((N_L1, D), jnp.float32),
    )(t6, l1c4, W1a, W1p4, b1f)


def _tc_final(vg_p, P4, Wp, W2f, b2f, Wc, bc):
    B = 2000

    def f(v_ref, p_ref, wp_ref, w2_ref, b2_ref, wc_ref, bc_ref, o_ref):
        pre = _unpack64(v_ref[...]) + jnp.dot(p_ref[...], wp_ref[...],
                                              preferred_element_type=jnp.float32)
        h = jnp.maximum(pre, 0.0)
        t7 = jnp.dot(h, w2_ref[...], preferred_element_type=jnp.float32) + b2_ref[...]
        o_ref[...] = jnp.dot(t7, wc_ref[...],
                             preferred_element_type=jnp.float32) + bc_ref[...]

    return pl.pallas_call(
        f,
        grid=(N_PTS // B,),
        in_specs=[pl.BlockSpec((B, D // 2), lambda i: (i, 0)),
                  pl.BlockSpec((B, 4), lambda i: (i, 0)),
                  _full(Wp), _full(W2f), _full(b2f), _full(Wc), _full(bc)],
        out_specs=pl.BlockSpec((B, NC), lambda i: (i, 0)),
        out_shape=jax.ShapeDtypeStruct((N_PTS, NC), jnp.float32),
    )(vg_p, P4, Wp, W2f, b2f, Wc, bc)


def _pad_rows(a, n):
    return jnp.pad(a, ((0, n - a.shape[0]), (0, 0)))


def _pad1(a, n, val=0):
    return jnp.pad(a, (0, n - a.shape[0]), constant_values=val)


def _bounds(sorted_ids, npt):
    cuts = (jnp.arange(33, dtype=jnp.int32) * npt).astype(jnp.int32)
    b = jnp.searchsorted(sorted_ids, cuts, side="left").astype(jnp.int32)
    return _pad1(b, 48)


def kernel(remission, points, l1_cluster_centers, l2_cluster_centers,
           l1_edges, l2_edges, l1_labels, l2_labels, params):
    f32 = jnp.float32
    l1c = l1_cluster_centers.astype(f32)
    l1c4 = jnp.pad(l1c, ((0, 0), (0, 1)))

    src = l1_edges[0].astype(jnp.int32)
    dst = l1_edges[1].astype(jnp.int32)
    perm1 = jnp.argsort(dst)
    src_s = _pad1(src[perm1], E1_PAD)
    dst_s = _pad1(dst[perm1], E1_PAD, val=NW * NP1 + 7)
    bounds1 = _bounds(dst_s[:E1], NP1)

    lab2 = l2_labels.astype(jnp.int32)
    perm_l2 = jnp.argsort(lab2)
    l2srt = lab2[perm_l2]
    perm_l2p = _pad1(perm_l2, L2M_PAD)
    l2srt_p = _pad1(l2srt, L2M_PAD, val=NW * NP2 + 7)
    bounds2 = _bounds(l2srt, NP2)

    src4 = l2_edges[0].astype(jnp.int32)
    dst4 = l2_edges[1].astype(jnp.int32)
    perm4 = jnp.argsort(dst4)
    src4_s = _pad1(src4[perm4], E2_PAD)
    dst4_s = _pad1(dst4[perm4], E2_PAD, val=NW * NP2 + 7)
    bounds4 = _bounds(dst4_s[:E2], NP2)

    lab1 = l1_labels.astype(jnp.int32)
    lab1_p = _pad1(lab1, PTS_PAD)
    lab1_fin = _pad1(lab1, FIN_PAD)
    l2lab_p = _pad1(lab2, UNPOOL_PAD)

    P4 = jnp.concatenate([remission.astype(f32), points.astype(f32)], axis=1)
    P8 = jnp.concatenate(
        [P4, jnp.ones((N_PTS, 1), f32), jnp.zeros((N_PTS, 3), f32)], axis=1)
    P8 = _pad_rows(P8, PTS_PAD).reshape(-1)

    p = params
    ffn = p["ffn"]

    def b2d(b):
        return b.reshape(1, -1).astype(f32)

    part = _sc_scatter_points(P8, lab1_p).reshape(NW, N_L1, 8)
    names = ["g2", "g2_1", "g2_2", "g2_3", "g6", "g6_1", "g6_2", "g6_3"]
    W1x = {n: p[n]["edge"]["W1"][:D].astype(f32) for n in names}
    W1r4 = {n: jnp.pad(p[n]["edge"]["W1"][D:].astype(f32), ((0, 1), (0, 0)))
            for n in names}
    t1, hx = _tc_combine_mlp1(part, l1c4, ffn["W1"].astype(f32), b2d(ffn["b1"]),
                              ffn["W2"].astype(f32), b2d(ffn["b2"]), W1x["g2"])

    rel4 = _sc_rel(l1c4.reshape(-1), src_s, jnp.clip(dst_s, 0, N_L1 - 1))

    def gnn(x, hx, name, skip=None, next_name=None):
        pe, po = p[name]["edge"], p[name]["out"]
        hsrc_p = _sc_gather(_pack32(hx), src_s, E1_PAD)
        ef = _tc_edge(hsrc_p, rel4, W1r4[name], b2d(pe["b1"]), pe["W2"].astype(f32))
        raw = _sc_segmax(ef, dst_s, bounds1, E1_PAD, NP1).reshape(-1, D)
        wxn = W1x[next_name] if next_name else None
        return _tc_node(raw[:N_L1], x, b2d(pe["b2"]), po["W1"].astype(f32),
                        b2d(po["b1"]), po["W2"].astype(f32), b2d(po["b2"]),
                        skip=skip, Wxn=wxn)

    t2, hx = gnn(t1, hx, "g2", next_name="g2_1")
    t2_1, hx = gnn(t2, hx, "g2_1", next_name="g2_2")
    t2_2, hx = gnn(t2_1, hx, "g2_2", next_name="g2_3")
    t2_3 = gnn(t2_2, hx, "g2_3")

    g23s = _sc_gather(t2_3, perm_l2p, L2M_PAD)
    raw3 = _sc_segmax(g23s, l2srt_p, bounds2, L2M_PAD, NP2).reshape(-1, D)
    t3 = _tc_finite0(raw3[:N_L2])
    g3 = _sc_gather(t3, src4_s, E2_PAD)
    raw4 = _sc_segmax(g3, dst4_s, bounds4, E2_PAD, NP2).reshape(-1, D)
    t4 = _tc_finite0(raw4[:N_L2], add=t3)
    t5 = _sc_gather(t4, l2lab_p, UNPOOL_PAD)[:N_L1]
    hx = _tc_matmul(t5, W1x["g6"], N_L1)

    t6, hx = gnn(t5, hx, "g6", skip=t2_3, next_name="g6_1")
    t6, hx = gnn(t6, hx, "g6_1", skip=t2_2, next_name="g6_2")
    t6, hx = gnn(t6, hx, "g6_2", skip=t2_1, next_name="g6_3")
    t6 = gnn(t6, hx, "g6_3", skip=t2)

    fb = p["fbn"]
    W1a = fb["W1"][:D].astype(f32)
    W1p = fb["W1"][D:D + 3].astype(f32)
    w1r = fb["W1"][D + 3].astype(f32)
    W1p4 = jnp.pad(W1p, ((0, 1), (0, 0)))
    v = _tc_vtable(t6, l1c4, W1a, W1p4, b2d(fb["b1"]))
    vg_p = _sc_gather(_pack32(v), lab1_fin, FIN_PAD)
    Wp = jnp.concatenate([w1r[None, :], W1p], axis=0)
    logits = _tc_final(vg_p[:N_PTS], P4, Wp, fb["W2"].astype(f32), b2d(fb["b2"]),
                       p["cls"]["W"].astype(f32), b2d(p["cls"]["b"]))
    return logits


# speedup vs baseline: 1.1651x; 1.0842x over previous
import dataclasses
import functools

import jax
import jax.numpy as jnp
from jax import lax
from jax.experimental import pallas as pl
from jax.experimental.pallas import tpu as pltpu
from jax.experimental.pallas import tpu_sc as plsc

N_PTS = 100000
N_L1 = 10000
N_L2 = 1000
E1 = 160000
E2 = 16000
D = 64
NC = 20

NCORES = 2
NSUB = 16
NW = NCORES * NSUB
LANES = 16

PTS_PAD = 100352
PTS_PER_TILE = PTS_PAD // NW
E1_PAD = 160768
NP1 = 313
L2M_PAD = 10240
NP2 = 32
E2_PAD = 16128
UNPOOL_PAD = 10112
FIN_PAD = 100096
CHUNK = 256


def _sc_params():
    cp = pltpu.CompilerParams()
    fields = pltpu.CompilerParams.__dataclass_fields__
    if "needs_layout_passes" in fields:
        cp = dataclasses.replace(cp, needs_layout_passes=False)
    if "use_tc_tiling_on_sc" in fields:
        cp = dataclasses.replace(cp, use_tc_tiling_on_sc=False)
    return cp


@functools.lru_cache(maxsize=1)
def _mesh():
    return plsc.VectorSubcoreMesh(core_axis_name="c", subcore_axis_name="s")


def _wid():
    return lax.axis_index("s") * NCORES + lax.axis_index("c")


def _sc_scatter_points(p8, lab):
    @functools.partial(
        pl.kernel,
        out_type=jax.ShapeDtypeStruct((NW, N_L1 * 8), jnp.float32),
        mesh=_mesh(),
        compiler_params=_sc_params(),
        scratch_types=[
            pltpu.VMEM((PTS_PER_TILE * 8,), jnp.float32),
            pltpu.VMEM((PTS_PER_TILE,), jnp.int32),
            pltpu.VMEM((N_L1 * 8,), jnp.float32),
        ],
    )
    def k(p8_hbm, lab_hbm, out_hbm, p_v, lab_v, acc_v):
        w = _wid()
        pltpu.sync_copy(p8_hbm.at[pl.ds(w * PTS_PER_TILE * 8, PTS_PER_TILE * 8)], p_v)
        pltpu.sync_copy(lab_hbm.at[pl.ds(w * PTS_PER_TILE, PTS_PER_TILE)], lab_v)

        zeros = jnp.zeros((LANES,), jnp.float32)

        @pl.loop(0, N_L1 * 8 // LANES)
        def _(i):
            acc_v[pl.ds(i * LANES, LANES)] = zeros

        iota = lax.iota(jnp.int32, LANES)

        @pl.loop(0, PTS_PER_TILE // LANES)
        def _(g):
            labs = lab_v[pl.ds(g * LANES, LANES)]
            base8 = labs * 8
            rows8 = (iota + g * LANES) * 8
            for f in range(5):
                vals = plsc.load_gather(p_v, [rows8 + f])
                plsc.addupdate_scatter(acc_v, [base8 + f], vals)

        pltpu.sync_copy(acc_v, out_hbm.at[w])

    return k(p8, lab)


def _sc_gather(table, idx, n_out):
    idx2 = idx.reshape(1, n_out)
    d = table.shape[1]
    dt = table.dtype

    @functools.partial(
        pl.kernel,
        out_type=jax.ShapeDtypeStruct((n_out, d), dt),
        mesh=_mesh(),
        compiler_params=_sc_params(),
    )
    def k(t_hbm, i_hbm, o_hbm):
        def body(i_vmem, o_vmem):
            pltpu.sync_copy(t_hbm.at[i_vmem.at[0]], o_vmem)

        pltpu.emit_pipeline(
            body,
            grid=(n_out // 128,),
            in_specs=[pl.BlockSpec((1, 128), index_map=lambda i: (0, i))],
            out_specs=[pl.BlockSpec((128, d), index_map=lambda i: (i, 0))],
            core_axis_name=("c", "s"),
            dimension_semantics=(pltpu.PARALLEL,),
        )(i_hbm, o_hbm)

    return k(table, idx2)


EPT = E1_PAD // NW


def _sc_rel(c4flat, src, dst):
    @functools.partial(
        pl.kernel,
        out_type=jax.ShapeDtypeStruct((E1_PAD * 4,), jnp.float32),
        mesh=_mesh(),
        compiler_params=_sc_params(),
        scratch_types=[
            pltpu.VMEM((N_L1 * 4,), jnp.float32),
            pltpu.VMEM((EPT,), jnp.int32),
            pltpu.VMEM((EPT,), jnp.int32),
            pltpu.VMEM((EPT * 4,), jnp.float32),
        ],
    )
    def k(c_hbm, s_hbm, d_hbm, o_hbm, c_v, s_v, d_v, o_v):
        w = _wid()
        pltpu.sync_copy(c_hbm, c_v)
        pltpu.sync_copy(s_hbm.at[pl.ds(w * EPT, EPT)], s_v)
        pltpu.sync_copy(d_hbm.at[pl.ds(w * EPT, EPT)], d_v)

        zeros = jnp.zeros((LANES,), jnp.float32)

        @pl.loop(0, EPT * 4 // LANES)
        def _(i):
            o_v[pl.ds(i * LANES, LANES)] = zeros

        iota4 = lax.iota(jnp.int32, LANES) * 4

        @pl.loop(0, EPT // LANES)
        def _(g):
            s4 = s_v[pl.ds(g * LANES, LANES)] * 4
            d4 = d_v[pl.ds(g * LANES, LANES)] * 4
            pos = iota4 + g * (LANES * 4)
            for c in range(3):
                a = plsc.load_gather(c_v, [s4 + c])
                b = plsc.load_gather(c_v, [d4 + c])
                plsc.store_scatter(o_v, [pos + c], a - b)

        pltpu.sync_copy(o_v, o_hbm.at[pl.ds(w * EPT * 4, EPT * 4)])

    return k(c4flat, src, dst).reshape(E1_PAD, 4)


def _sc_segmax(vals, dst, bounds, e_pad, npt):
    n_out = NW * npt

    @functools.partial(
        pl.kernel,
        out_type=jax.ShapeDtypeStruct((n_out * D,), jnp.float32),
        mesh=_mesh(),
        compiler_params=_sc_params(),
        scratch_types=[
            pltpu.VMEM((npt * D,), jnp.float32),
            pltpu.VMEM((2 * CHUNK, D), jnp.float32),
            pltpu.VMEM((2 * CHUNK,), jnp.int32),
            pltpu.VMEM((48,), jnp.int32),
            pltpu.SemaphoreType.DMA,
        ],
    )
    def k(v_hbm, d_hbm, b_hbm, o_hbm, acc_v, buf_v, dbuf_v, b_v, sem):
        w = _wid()
        pltpu.sync_copy(b_hbm, b_v)
        wsplat = jnp.full((LANES,), w, jnp.int32)
        lo = plsc.load_gather(b_v, [wsplat])[0]
        hi = plsc.load_gather(b_v, [wsplat + 1])[0]
        base = w * npt

        ninf = jnp.full((LANES,), -jnp.inf, jnp.float32)

        @pl.loop(0, npt * D // LANES)
        def _(i):
            acc_v[pl.ds(i * LANES, LANES)] = ninf

        k0 = lo // CHUNK
        nchunks = (hi + CHUNK - 1) // CHUNK - k0

        def issue(kk, par):
            s = (k0 + kk) * CHUNK
            pltpu.async_copy(v_hbm.at[pl.ds(s, CHUNK)],
                             buf_v.at[pl.ds(par * CHUNK, CHUNK)], sem)
            pltpu.async_copy(d_hbm.at[pl.ds(s, CHUNK)],
                             dbuf_v.at[pl.ds(par * CHUNK, CHUNK)], sem)

        @pl.when(nchunks > 0)
        def _():
            issue(0, 0)

        @pl.loop(0, nchunks)
        def _(kk):
            par = lax.rem(kk, 2)
            off = par * CHUNK
            pltpu.make_async_copy(v_hbm.at[pl.ds(0, CHUNK)],
                                  buf_v.at[pl.ds(off, CHUNK)], sem).wait()
            pltpu.make_async_copy(d_hbm.at[pl.ds(0, CHUNK)],
                                  dbuf_v.at[pl.ds(off, CHUNK)], sem).wait()

            @pl.when(kk + 1 < nchunks)
            def _():
                issue(kk + 1, 1 - par)

            start = (k0 + kk) * CHUNK
            interior = jnp.logical_and(start >= lo, start + CHUNK <= hi)

            @pl.when(interior)
            def _():
                @pl.loop(0, CHUNK // LANES)
                def _(g):
                    dvec = dbuf_v[pl.ds(off + g * LANES, LANES)]
                    for l in range(LANES):
                        i = off + g * LANES + l
                        dl = dvec[l] - base
                        for j in range(D // LANES):
                            v = buf_v[i, pl.ds(j * LANES, LANES)]
                            o2 = dl * D + j * LANES
                            acc_v[pl.ds(o2, LANES)] = jnp.maximum(
                                acc_v[pl.ds(o2, LANES)], v)

            @pl.when(jnp.logical_not(interior))
            def _():
                @pl.loop(0, CHUNK // LANES)
                def _(g):
                    dvec = dbuf_v[pl.ds(off + g * LANES, LANES)]
                    for l in range(LANES):
                        i = g * LANES + l
                        gi = start + i
                        ok = jnp.logical_and(gi >= lo, gi < hi)
                        dl = jnp.where(ok, dvec[l] - base, 0)
                        for j in range(D // LANES):
                            v = buf_v[off + i, pl.ds(j * LANES, LANES)]
                            v = jnp.where(ok, v, -jnp.inf)
                            o2 = dl * D + j * LANES
                            acc_v[pl.ds(o2, LANES)] = jnp.maximum(
                                acc_v[pl.ds(o2, LANES)], v)

        pltpu.sync_copy(acc_v, o_hbm.at[pl.ds(base * D, npt * D)])

    return k(vals, dst, bounds)


def _full(a):
    return pl.BlockSpec(a.shape, lambda *_: tuple(0 for _ in a.shape))


def _tc_combine_mlp1(part, l1c4, W1, b1, W2, b2, W1x):
    B = 1000

    def f(p_ref, c_ref, w1_ref, b1_ref, w2_ref, b2_ref, wx_ref, t1_ref, hx_ref):
        S = jnp.sum(p_ref[...], axis=0)
        agg = jnp.concatenate(
            [S[:, 0:1], S[:, 1:4] - S[:, 4:5] * c_ref[...][:, 0:3]], axis=1)
        h = jnp.maximum(jnp.dot(agg, w1_ref[...],
                                preferred_element_type=jnp.float32) + b1_ref[...], 0.0)
        t1 = jnp.dot(h, w2_ref[...], preferred_element_type=jnp.float32) + b2_ref[...]
        t1_ref[...] = t1
        hx_ref[...] = jnp.dot(t1, wx_ref[...], preferred_element_type=jnp.float32)

    return pl.pallas_call(
        f,
        grid=(N_L1 // B,),
        in_specs=[
            pl.BlockSpec((NW, B, 8), lambda i: (0, i, 0)),
            pl.BlockSpec((B, 4), lambda i: (i, 0)),
            _full(W1), _full(b1), _full(W2), _full(b2), _full(W1x),
        ],
        out_specs=[pl.BlockSpec((B, D), lambda i: (i, 0)),
                   pl.BlockSpec((B, D), lambda i: (i, 0))],
        out_shape=[jax.ShapeDtypeStruct((N_L1, D), jnp.float32),
                   jax.ShapeDtypeStruct((N_L1, D), jnp.float32)],
    )(part, l1c4, W1, b1, W2, b2, W1x)


def _pack32(h):
    lo = lax.bitcast_convert_type(h[:, :32].astype(jnp.bfloat16),
                                  jnp.uint16).astype(jnp.uint32)
    hi = lax.bitcast_convert_type(h[:, 32:].astype(jnp.bfloat16),
                                  jnp.uint16).astype(jnp.uint32)
    return lax.bitcast_convert_type(lo | (hi << 16), jnp.int32)


def _unpack64(xi):
    lo = lax.bitcast_convert_type(jnp.left_shift(xi, 16), jnp.float32)
    hi = lax.bitcast_convert_type(
        jnp.bitwise_and(xi, jnp.int32(-65536)), jnp.float32)
    return jnp.concatenate([lo, hi], axis=1)


def _tc_edge(hsrc_p, rel4, W1r4, b1, W2):
    B = 1024

    def f(h_ref, r_ref, wr_ref, b1_ref, w2_ref, o_ref):
        pre = _unpack64(h_ref[...]) + jnp.dot(
            r_ref[...], wr_ref[...],
            preferred_element_type=jnp.float32) + b1_ref[...]
        o_ref[...] = jnp.dot(jnp.maximum(pre, 0.0), w2_ref[...],
                             preferred_element_type=jnp.float32)

    return pl.pallas_call(
        f,
        grid=(E1_PAD // B,),
        in_specs=[pl.BlockSpec((B, D // 2), lambda i: (i, 0)),
                  pl.BlockSpec((B, 4), lambda i: (i, 0)),
                  _full(W1r4), _full(b1), _full(W2)],
        out_specs=pl.BlockSpec((B, D), lambda i: (i, 0)),
        out_shape=jax.ShapeDtypeStruct((E1_PAD, D), jnp.float32),
    )(hsrc_p, rel4, W1r4, b1, W2)


def _tc_node(raw, x, b2e, Wo1, bo1, Wo2, bo2, skip=None, Wxn=None):
    B = 1000
    n_in = [raw, x, b2e, Wo1, bo1, Wo2, bo2]
    specs = [pl.BlockSpec((B, D), lambda i: (i, 0)),
             pl.BlockSpec((B, D), lambda i: (i, 0)),
             _full(b2e), _full(Wo1), _full(bo1), _full(Wo2), _full(bo2)]
    if skip is not None:
        n_in.append(skip)
        specs.append(pl.BlockSpec((B, D), lambda i: (i, 0)))
    if Wxn is not None:
        n_in.append(Wxn)
        specs.append(_full(Wxn))

    has_skip = skip is not None
    has_wxn = Wxn is not None

    def f(*refs):
        raw_r, x_r, b2_r, w1_r, b1_r, w2_r, bo2_r = refs[:7]
        pos = 7
        rawv = raw_r[...]
        agg = jnp.where(jnp.isfinite(rawv), rawv + b2_r[...], 0.0)
        h = jnp.maximum(jnp.dot(agg, w1_r[...],
                                preferred_element_type=jnp.float32) + b1_r[...], 0.0)
        y = x_r[...] + jnp.dot(h, w2_r[...],
                               preferred_element_type=jnp.float32) + bo2_r[...]
        if has_skip:
            y = y + refs[pos][...]
            pos += 1
        if has_wxn:
            wx_r = refs[pos]
            y_ref, hx_ref = refs[-2:]
            y_ref[...] = y
            hx_ref[...] = jnp.dot(y, wx_r[...], preferred_element_type=jnp.float32)
        else:
            refs[-1][...] = y

    out_specs = pl.BlockSpec((B, D), lambda i: (i, 0))
    out_shape = jax.ShapeDtypeStruct((N_L1, D), jnp.float32)
    if has_wxn:
        out_specs = [out_specs, pl.BlockSpec((B, D), lambda i: (i, 0))]
        out_shape = [out_shape, jax.ShapeDtypeStruct((N_L1, D), jnp.float32)]

    return pl.pallas_call(
        f,
        grid=(N_L1 // B,),
        in_specs=specs,
        out_specs=out_specs,
        out_shape=out_shape,
    )(*n_in)


def _tc_matmul(x, W, n_rows):
    B = 1000

    def f(x_ref, w_ref, o_ref):
        o_ref[...] = jnp.dot(x_ref[...], w_ref[...],
                             preferred_element_type=jnp.float32)

    return pl.pallas_call(
        f,
        grid=(n_rows // B,),
        in_specs=[pl.BlockSpec((B, D), lambda i: (i, 0)), _full(W)],
        out_specs=pl.BlockSpec((B, D), lambda i: (i, 0)),
        out_shape=jax.ShapeDtypeStruct((n_rows, D), jnp.float32),
    )(x, W)


def _tc_finite0(raw, add=None):
    has_add = add is not None

    def f(*refs):
        r = refs[0][...]
        t = jnp.where(jnp.isfinite(r), r, 0.0)
        if has_add:
            t = t + refs[1][...]
        refs[-1][...] = t

    n_in = [raw] + ([add] if has_add else [])
    specs = [pl.BlockSpec((N_L2, D), lambda: (0, 0))]
    if has_add:
        specs.append(pl.BlockSpec((N_L2, D), lambda: (0, 0)))
    return pl.pallas_call(
        f,
        grid=(),
        in_specs=specs,
        out_specs=pl.BlockSpec((N_L2, D), lambda: (0, 0)),
        out_shape=jax.ShapeDtypeStruct((N_L2, D), jnp.float32),
    )(*n_in)


def _tc_vtable(t6, l1c4, W1a, W1p4, b1f):
    B = 1000

    def f(t_ref, c_ref, wa_ref, wp_ref, b_ref, o_ref):
        o_ref[...] = (jnp.dot(t_ref[...], wa_ref[...],
                              preferred_element_type=jnp.float32)
                      - jnp.dot(c_ref[...], wp_ref[...],
                                preferred_element_type=jnp.float32)
                      + b_ref[...])

    return pl.pallas_call(
        f,
        grid=(N_L1 // B,),
        in_specs=[pl.BlockSpec((B, D), lambda i: (i, 0)),
                  pl.BlockSpec((B, 4), lambda i: (i, 0)),
                  _full(W1a), _full(W1p4), _full(b1f)],
        out_specs=pl.BlockSpec((B, D), lambda i: (i, 0)),
        out_shape=jax.ShapeDtypeStruct((N_L1, D), jnp.float32),
    )(t6, l1c4, W1a, W1p4, b1f)


def _tc_final(vg_p, P4, Wp, W2f, b2f, Wc, bc):
    B = 2000

    def f(v_ref, p_ref, wp_ref, w2_ref, b2_ref, wc_ref, bc_ref, o_ref):
        pre = _unpack64(v_ref[...]) + jnp.dot(p_ref[...], wp_ref[...],
                                              preferred_element_type=jnp.float32)
        h = jnp.maximum(pre, 0.0)
        t7 = jnp.dot(h, w2_ref[...], preferred_element_type=jnp.float32) + b2_ref[...]
        o_ref[...] = jnp.dot(t7, wc_ref[...],
                             preferred_element_type=jnp.float32) + bc_ref[...]

    return pl.pallas_call(
        f,
        grid=(N_PTS // B,),
        in_specs=[pl.BlockSpec((B, D // 2), lambda i: (i, 0)),
                  pl.BlockSpec((B, 4), lambda i: (i, 0)),
                  _full(Wp), _full(W2f), _full(b2f), _full(Wc), _full(bc)],
        out_specs=pl.BlockSpec((B, NC), lambda i: (i, 0)),
        out_shape=jax.ShapeDtypeStruct((N_PTS, NC), jnp.float32),
    )(vg_p, P4, Wp, W2f, b2f, Wc, bc)


def _pad_rows(a, n):
    return jnp.pad(a, ((0, n - a.shape[0]), (0, 0)))


def _pad1(a, n, val=0):
    return jnp.pad(a, (0, n - a.shape[0]), constant_values=val)


def _bounds(sorted_ids, npt):
    cuts = (jnp.arange(33, dtype=jnp.int32) * npt).astype(jnp.int32)
    b = jnp.searchsorted(sorted_ids, cuts, side="left").astype(jnp.int32)
    return _pad1(b, 48)


def kernel(remission, points, l1_cluster_centers, l2_cluster_centers,
           l1_edges, l2_edges, l1_labels, l2_labels, params):
    f32 = jnp.float32
    l1c = l1_cluster_centers.astype(f32)
    l1c4 = jnp.pad(l1c, ((0, 0), (0, 1)))

    src = l1_edges[0].astype(jnp.int32)
    dst = l1_edges[1].astype(jnp.int32)
    perm1 = jnp.argsort(dst)
    src_s = _pad1(src[perm1], E1_PAD)
    dst_s = _pad1(dst[perm1], E1_PAD, val=NW * NP1 + 7)
    bounds1 = _bounds(dst_s[:E1], NP1)

    lab2 = l2_labels.astype(jnp.int32)
    perm_l2 = jnp.argsort(lab2)
    l2srt = lab2[perm_l2]
    perm_l2p = _pad1(perm_l2, L2M_PAD)
    l2srt_p = _pad1(l2srt, L2M_PAD, val=NW * NP2 + 7)
    bounds2 = _bounds(l2srt, NP2)

    src4 = l2_edges[0].astype(jnp.int32)
    dst4 = l2_edges[1].astype(jnp.int32)
    perm4 = jnp.argsort(dst4)
    src4_s = _pad1(src4[perm4], E2_PAD)
    dst4_s = _pad1(dst4[perm4], E2_PAD, val=NW * NP2 + 7)
    bounds4 = _bounds(dst4_s[:E2], NP2)

    lab1 = l1_labels.astype(jnp.int32)
    lab1_p = _pad1(lab1, PTS_PAD)
    lab1_fin = _pad1(lab1, FIN_PAD)
    l2lab_p = _pad1(lab2, UNPOOL_PAD)

    P4 = jnp.concatenate([remission.astype(f32), points.astype(f32)], axis=1)
    P8 = jnp.concatenate(
        [P4, jnp.ones((N_PTS, 1), f32), jnp.zeros((N_PTS, 3), f32)], axis=1)
    P8 = _pad_rows(P8, PTS_PAD).reshape(-1)

    p = params
    ffn = p["ffn"]

    def b2d(b):
        return b.reshape(1, -1).astype(f32)

    part = _sc_scatter_points(P8, lab1_p).reshape(NW, N_L1, 8)
    names = ["g2", "g2_1", "g2_2", "g2_3", "g6", "g6_1", "g6_2", "g6_3"]
    W1x = {n: p[n]["edge"]["W1"][:D].astype(f32) for n in names}
    W1r4 = {n: jnp.pad(p[n]["edge"]["W1"][D:].astype(f32), ((0, 1), (0, 0)))
            for n in names}
    t1, hx = _tc_combine_mlp1(part, l1c4, ffn["W1"].astype(f32), b2d(ffn["b1"]),
                              ffn["W2"].astype(f32), b2d(ffn["b2"]), W1x["g2"])

    rel4 = _sc_rel(l1c4.reshape(-1), src_s, jnp.clip(dst_s, 0, N_L1 - 1))

    def gnn(x, hx, name, skip=None, next_name=None):
        pe, po = p[name]["edge"], p[name]["out"]
        hsrc_p = _sc_gather(_pack32(hx), src_s, E1_PAD)
        ef = _tc_edge(hsrc_p, rel4, W1r4[name], b2d(pe["b1"]), pe["W2"].astype(f32))
        raw = _sc_segmax(ef, dst_s, bounds1, E1_PAD, NP1).reshape(-1, D)
        wxn = W1x[next_name] if next_name else None
        return _tc_node(raw[:N_L1], x, b2d(pe["b2"]), po["W1"].astype(f32),
                        b2d(po["b1"]), po["W2"].astype(f32), b2d(po["b2"]),
                        skip=skip, Wxn=wxn)

    t2, hx = gnn(t1, hx, "g2", next_name="g2_1")
    t2_1, hx = gnn(t2, hx, "g2_1", next_name="g2_2")
    t2_2, hx = gnn(t2_1, hx, "g2_2", next_name="g2_3")
    t2_3 = gnn(t2_2, hx, "g2_3")

    g23s = _sc_gather(t2_3, perm_l2p, L2M_PAD)
    raw3 = _sc_segmax(g23s, l2srt_p, bounds2, L2M_PAD, NP2).reshape(-1, D)
    t3 = _tc_finite0(raw3[:N_L2])
    g3 = _sc_gather(t3, src4_s, E2_PAD)
    raw4 = _sc_segmax(g3, dst4_s, bounds4, E2_PAD, NP2).reshape(-1, D)
    t4 = _tc_finite0(raw4[:N_L2], add=t3)
    t5 = _sc_gather(t4, l2lab_p, UNPOOL_PAD)[:N_L1]
    hx = _tc_matmul(t5, W1x["g6"], N_L1)

    t6, hx = gnn(t5, hx, "g6", skip=t2_3, next_name="g6_1")
    t6, hx = gnn(t6, hx, "g6_1", skip=t2_2, next_name="g6_2")
    t6, hx = gnn(t6, hx, "g6_2", skip=t2_1, next_name="g6_3")
    t6 = gnn(t6, hx, "g6_3", skip=t2)

    fb = p["fbn"]
    W1a = fb["W1"][:D].astype(f32)
    W1p = fb["W1"][D:D + 3].astype(f32)
    w1r = fb["W1"][D + 3].astype(f32)
    W1p4 = jnp.pad(W1p, ((0, 1), (0, 0)))
    v = _tc_vtable(t6, l1c4, W1a, W1p4, b2d(fb["b1"]))
    vg_p = _sc_gather(_pack32(v), lab1_fin, FIN_PAD)
    Wp = jnp.concatenate([w1r[None, :], W1p], axis=0)
    logits = _tc_final(vg_p[:N_PTS], P4, Wp, fb["W2"].astype(f32), b2d(fb["b2"]),
                       p["cls"]["W"].astype(f32), b2d(p["cls"]["b"]))
    return logits
